# Initial kernel scaffold; baseline (speedup 1.0000x reference)
import jax
import jax.numpy as jnp
from jax.experimental import pallas as pl


def kernel(x, edge_index, W1, b1, W2, b2):
    raise NotImplementedError("write your pallas kernel here")


# speedup vs baseline: 55.1404x; 55.1404x over previous
import functools

import jax
import jax.numpy as jnp
from jax import lax
from jax.experimental import pallas as pl
from jax.experimental.pallas import tpu as pltpu
from jax.experimental.pallas import tpu_sc as plsc

N = 10000
E = 320000
L = 16
NC = 2
NS = 16
NW = NC * NS
E_PER_W = E // NW
BLOCK = 2000
G = N // BLOCK


def _presence_body(src_hbm, out_hbm, idx_v, mark_v):
    wid = lax.axis_index("s") * NC + lax.axis_index("c")
    pltpu.sync_copy(src_hbm.at[pl.ds(wid * E_PER_W, E_PER_W)], idx_v)

    zeros = jnp.zeros((L,), jnp.float32)
    ones = jnp.ones((L,), jnp.float32)

    def zero_body(i, _):
        mark_v[pl.ds(i * L, L)] = zeros
        return ()

    lax.fori_loop(0, N // L, zero_body, (), unroll=4)

    def scatter_body(i, _):
        idx = idx_v[pl.ds(i * L, L)]
        plsc.store_scatter(mark_v, [idx], ones)
        return ()

    lax.fori_loop(0, E_PER_W // L, scatter_body, (), unroll=4)

    for g in range(G):
        pltpu.sync_copy(
            mark_v.at[pl.ds(g * BLOCK, BLOCK)],
            out_hbm.at[pl.ds(g * NW * BLOCK + wid * BLOCK, BLOCK)],
        )


_presence_kernel = functools.partial(
    pl.kernel,
    out_type=jax.ShapeDtypeStruct((G * NW * BLOCK,), jnp.float32),
    mesh=plsc.VectorSubcoreMesh(core_axis_name="c", subcore_axis_name="s"),
    scratch_types=[
        pltpu.VMEM((E_PER_W,), jnp.int32),
        pltpu.VMEM((N,), jnp.float32),
    ],
    compiler_params=pltpu.CompilerParams(needs_layout_passes=False),
)(_presence_body)


def _dense_body(x_ref, pres_ref, w1_ref, b1_ref, w2_ref, b2_ref, out_ref):
    deg = jnp.sum(pres_ref[0], axis=0)
    m = (deg > 0.0).astype(jnp.float32)[:, None]
    z1 = (
        jnp.dot(x_ref[...], w1_ref[...], preferred_element_type=jnp.float32)
        + b1_ref[...]
    )
    h1 = jnp.maximum(z1, 0.0) * m
    z2 = (
        jnp.dot(h1, w2_ref[...], preferred_element_type=jnp.float32)
        + b2_ref[...]
    ) * m
    zmax = jnp.max(z2, axis=1, keepdims=True)
    zs = z2 - zmax
    lse = jnp.log(jnp.sum(jnp.exp(zs), axis=1, keepdims=True))
    out_ref[...] = zs - lse


def _dense_call(x, pres, W1, b1, W2, b2, block):
    grid = (N // block,)
    return pl.pallas_call(
        _dense_body,
        grid=grid,
        in_specs=[
            pl.BlockSpec((block, 128), lambda j: (j, 0)),
            pl.BlockSpec((1, NW, block), lambda j: (j, 0, 0)),
            pl.BlockSpec((128, 128), lambda j: (0, 0)),
            pl.BlockSpec((1, 128), lambda j: (0, 0)),
            pl.BlockSpec((128, 64), lambda j: (0, 0)),
            pl.BlockSpec((1, 64), lambda j: (0, 0)),
        ],
        out_specs=pl.BlockSpec((block, 64), lambda j: (j, 0)),
        out_shape=jax.ShapeDtypeStruct((N, 64), jnp.float32),
    )(x, pres, W1, b1.reshape(1, 128), W2, b2.reshape(1, 64))


def kernel(x, edge_index, W1, b1, W2, b2):
    source = edge_index[0].astype(jnp.int32)
    pres = _presence_kernel(source).reshape(G, NW, BLOCK)
    return _dense_call(x, pres, W1, b1, W2, b2, block=BLOCK)


# speedup vs baseline: 76.6499x; 1.3901x over previous
import functools

import jax
import jax.numpy as jnp
from jax import lax
from jax.experimental import pallas as pl
from jax.experimental.pallas import tpu as pltpu
from jax.experimental.pallas import tpu_sc as plsc

N = 10000
E = 320000
L = 16
NC = 2
NS = 16
NW = NC * NS
E_PER_W = E // NW
BLOCK = 2000
G = N // BLOCK


T128 = 128
TILES = E // T128
BASE_TILES = TILES // NW
EXTRA = TILES - BASE_TILES * NW
MAX_EDGES_W = (BASE_TILES + 1) * T128


def _presence_body(src_hbm, out_hbm, idx_v, mark_v):
    wid = lax.axis_index("s") * NC + lax.axis_index("c")
    extra_before = jnp.maximum(wid - (NW - EXTRA), 0)
    t0 = wid * BASE_TILES + extra_before

    zeros = jnp.zeros((L,), jnp.float32)
    ones = jnp.ones((L,), jnp.float32)

    @plsc.parallel_loop(0, N, step=L, unroll=8)
    def _zero(i):
        mark_v[pl.ds(i, L)] = zeros

    def _mark_edges(nt):
        ne = nt * T128
        pltpu.sync_copy(
            src_hbm.at[:, pl.ds(pl.multiple_of(t0 * T128, T128), ne)],
            idx_v.at[:, pl.ds(0, ne)],
        )

        @plsc.parallel_loop(0, ne, step=L, unroll=8)
        def _scatter(i):
            idx = idx_v[0, pl.ds(i, L)]
            plsc.store_scatter(mark_v, [idx], ones)

    @pl.when(wid >= NW - EXTRA)
    def _():
        _mark_edges(BASE_TILES + 1)

    @pl.when(wid < NW - EXTRA)
    def _():
        _mark_edges(BASE_TILES)

    for g in range(G):
        pltpu.sync_copy(
            mark_v.at[pl.ds(g * BLOCK, BLOCK)],
            out_hbm.at[pl.ds(g * NW * BLOCK + wid * BLOCK, BLOCK)],
        )


_presence_kernel = functools.partial(
    pl.kernel,
    out_type=jax.ShapeDtypeStruct((G * NW * BLOCK,), jnp.float32),
    mesh=plsc.VectorSubcoreMesh(core_axis_name="c", subcore_axis_name="s"),
    scratch_types=[
        pltpu.VMEM((2, MAX_EDGES_W), jnp.int32),
        pltpu.VMEM((N,), jnp.float32),
    ],
    compiler_params=pltpu.CompilerParams(needs_layout_passes=False),
)(_presence_body)


def _dense_body(x_ref, pres_ref, w1_ref, b1_ref, w2_ref, b2_ref, out_ref):
    deg = jnp.sum(pres_ref[0], axis=0)
    m = (deg > 0.0).astype(jnp.float32)[:, None]
    z1 = (
        jnp.dot(x_ref[...], w1_ref[...], preferred_element_type=jnp.float32)
        + b1_ref[...]
    )
    h1 = jnp.maximum(z1, 0.0)
    z2 = (
        jnp.dot(h1, w2_ref[...], preferred_element_type=jnp.float32)
        + b2_ref[...]
    ) * m
    zmax = jnp.max(z2, axis=1, keepdims=True)
    zs = z2 - zmax
    lse = jnp.log(jnp.sum(jnp.exp(zs), axis=1, keepdims=True))
    out_ref[...] = zs - lse


def _dense_call(x, pres, W1, b1, W2, b2, block):
    grid = (N // block,)
    return pl.pallas_call(
        _dense_body,
        grid=grid,
        in_specs=[
            pl.BlockSpec((block, 128), lambda j: (j, 0)),
            pl.BlockSpec((1, NW, block), lambda j: (j, 0, 0)),
            pl.BlockSpec((128, 128), lambda j: (0, 0)),
            pl.BlockSpec((1, 128), lambda j: (0, 0)),
            pl.BlockSpec((128, 64), lambda j: (0, 0)),
            pl.BlockSpec((1, 64), lambda j: (0, 0)),
        ],
        out_specs=pl.BlockSpec((block, 64), lambda j: (j, 0)),
        out_shape=jax.ShapeDtypeStruct((N, 64), jnp.float32),
    )(x, pres, W1, b1.reshape(1, 128), W2, b2.reshape(1, 64))


def kernel(x, edge_index, W1, b1, W2, b2):
    source = edge_index.astype(jnp.int32)
    pres = _presence_kernel(source).reshape(G, NW, BLOCK)
    return _dense_call(x, pres, W1, b1, W2, b2, block=BLOCK)
